# Initial kernel scaffold; baseline (speedup 1.0000x reference)
import jax
import jax.numpy as jnp
from jax.experimental import pallas as pl


def kernel(x, hyperplanes, expert_weights):
    raise NotImplementedError("write your pallas kernel here")


# speedup vs baseline: 4.8567x; 4.8567x over previous
import jax
import jax.numpy as jnp
from jax import lax
from jax.experimental import pallas as pl
from jax.experimental.pallas import tpu as pltpu

_CHUNK = 128
_NBITS = 4
_NTILE = 512


def _route_kernel(x_ref, hp_ref, ids_ref):
    nc = x_ref.shape[0]
    emb = jnp.mean(x_ref[...], axis=1)
    proj = jnp.dot(emb, hp_ref[...],
                   preferred_element_type=jnp.float32)
    bits = (proj > 0).astype(jnp.int32)
    col = lax.broadcasted_iota(jnp.int32, (nc, _NBITS), 1)
    powers = jnp.left_shift(jnp.ones((nc, _NBITS), jnp.int32), col)
    e1 = jnp.sum(bits * powers, axis=1, keepdims=True)
    ap = jnp.abs(proj)
    mn = jnp.min(ap, axis=1, keepdims=True)
    cand = jnp.where(ap == mn, col, _NBITS)
    weak = jnp.min(cand, axis=1, keepdims=True)
    flip = jnp.left_shift(jnp.ones_like(weak), weak)
    e2 = jnp.bitwise_xor(e1, flip)
    ids_ref[...] = jnp.concatenate([e1, e2], axis=1)


def _ffn_kernel(ids_ref, x_ref, w1_ref, w2_ref, o_ref):
    del ids_ref
    xb = x_ref[0]
    wb = (w1_ref[0] + w2_ref[0]) * 0.5
    o_ref[0] = jnp.dot(xb, wb, preferred_element_type=jnp.float32)


def kernel(x, hyperplanes, expert_weights):
    bsz, seq, d = x.shape
    nc = (bsz * seq) // _CHUNK
    x3 = x.reshape(nc, _CHUNK, d)

    ids = pl.pallas_call(
        _route_kernel,
        out_shape=jax.ShapeDtypeStruct((nc, 2), jnp.int32),
    )(x3, hyperplanes)

    nt = d // _NTILE
    grid_spec = pltpu.PrefetchScalarGridSpec(
        num_scalar_prefetch=1,
        grid=(nc, nt),
        in_specs=[
            pl.BlockSpec((1, _CHUNK, d), lambda c, n, ids: (c, 0, 0)),
            pl.BlockSpec((1, d, _NTILE), lambda c, n, ids: (ids[c, 0], 0, n)),
            pl.BlockSpec((1, d, _NTILE), lambda c, n, ids: (ids[c, 1], 0, n)),
        ],
        out_specs=pl.BlockSpec((1, _CHUNK, _NTILE), lambda c, n, ids: (c, 0, n)),
    )
    out = pl.pallas_call(
        _ffn_kernel,
        grid_spec=grid_spec,
        out_shape=jax.ShapeDtypeStruct((nc, _CHUNK, d), jnp.float32),
    )(ids, x3, expert_weights, expert_weights)
    return out.reshape(bsz, seq, d)


# speedup vs baseline: 5.9644x; 1.2281x over previous
import jax
import jax.numpy as jnp
from jax import lax
from jax.experimental import pallas as pl
from jax.experimental.pallas import tpu as pltpu

_CHUNK = 128
_NBITS = 4
_NTILE = 512


def _route_kernel(x_ref, hp_ref, ids_ref):
    nc = x_ref.shape[0]
    emb = jnp.mean(x_ref[...], axis=1)
    proj = jnp.dot(emb, hp_ref[...],
                   preferred_element_type=jnp.float32)
    bits = (proj > 0).astype(jnp.int32)
    col = lax.broadcasted_iota(jnp.int32, (nc, _NBITS), 1)
    powers = jnp.left_shift(jnp.ones((nc, _NBITS), jnp.int32), col)
    e1 = jnp.sum(bits * powers, axis=1, keepdims=True)
    ap = jnp.abs(proj)
    mn = jnp.min(ap, axis=1, keepdims=True)
    cand = jnp.where(ap == mn, col, _NBITS)
    weak = jnp.min(cand, axis=1, keepdims=True)
    flip = jnp.left_shift(jnp.ones_like(weak), weak)
    e2 = jnp.bitwise_xor(e1, flip)
    ids_ref[...] = jnp.concatenate([e1, e2], axis=1)


def kernel(x, hyperplanes, expert_weights):
    bsz, seq, d = x.shape
    nc = (bsz * seq) // _CHUNK
    x3 = x.reshape(nc, _CHUNK, d)

    ids = pl.pallas_call(
        _route_kernel,
        out_shape=jax.ShapeDtypeStruct((nc, 2), jnp.int32),
    )(x3, hyperplanes)

    eflat = ids.reshape(-1)
    order = jnp.argsort(eflat)
    meta = jnp.stack([eflat[order], (order // 2).astype(jnp.int32)], axis=1)

    na = 2 * nc
    nt = d // _NTILE
    rows = nc * _CHUNK
    x2 = x.reshape(rows, d)

    def _moe_kernel(meta_ref, x_hbm, w_ref, o_ref, xs_ref, sem):
        n = pl.program_id(0)
        a = pl.program_id(1)

        @pl.when((n == 0) & (a == 0))
        def _stage_x():
            cp = pltpu.make_async_copy(x_hbm, xs_ref, sem)
            cp.start()
            cp.wait()

        @pl.when(a == 0)
        def _zero():
            o_ref[...] = jnp.zeros_like(o_ref)

        c = meta_ref[a, 1]
        xs = xs_ref[pl.ds(c * _CHUNK, _CHUNK), :]
        o_ref[pl.ds(c * _CHUNK, _CHUNK), :] += jnp.dot(
            xs, w_ref[0], preferred_element_type=jnp.float32) * 0.5

    grid_spec = pltpu.PrefetchScalarGridSpec(
        num_scalar_prefetch=1,
        grid=(nt, na),
        in_specs=[
            pl.BlockSpec(memory_space=pl.ANY),
            pl.BlockSpec((1, d, _NTILE), lambda n, a, meta: (meta[a, 0], 0, n)),
        ],
        out_specs=pl.BlockSpec((rows, _NTILE), lambda n, a, meta: (0, n)),
        scratch_shapes=[
            pltpu.VMEM((rows, d), jnp.float32),
            pltpu.SemaphoreType.DMA,
        ],
    )
    out = pl.pallas_call(
        _moe_kernel,
        grid_spec=grid_spec,
        out_shape=jax.ShapeDtypeStruct((rows, d), jnp.float32),
    )(meta, x2, expert_weights)
    return out.reshape(bsz, seq, d)
